# baseline (speedup 1.0000x reference)
import jax
import jax.numpy as jnp
from jax.experimental import pallas as pl
from jax.experimental.pallas import tpu as pltpu


def _fused_kernel(T, Cin, Cout, K, N, G, BB,
                  x_ref, wcat_ref, bcat_ref, lst_ref, wkcat_ref, bfg_ref,
                  o_ref):
    wcat = wcat_ref[...]
    bcat = bcat_ref[...]
    lst = lst_ref[...]
    bfg = bfg_ref[...]
    NG = T // G

    m = []
    for b in range(BB):
        xs = x_ref[b]
        zblk = jnp.zeros((Cin, N), xs.dtype)
        for g in range(NG):
            cols = []
            for i in range(G):
                t = g * G + i
                if t == 0:
                    cols.append(jnp.concatenate([zblk, xs[:2 * Cin]], axis=0))
                elif t == T - 1:
                    cols.append(jnp.concatenate([xs[(T - 2) * Cin:], zblk],
                                                axis=0))
                else:
                    cols.append(xs[(t - 1) * Cin:(t + 2) * Cin])
            x3g = jnp.concatenate(cols, axis=1)
            m.append(jnp.dot(wcat, x3g,
                             preferred_element_type=jnp.float32) + bcat)

    u = []
    for mg in m:
        x1g = jnp.concatenate(
            [mg[Cout:, i * N:(i + 1) * N] for i in range(G)],
            axis=0).astype(jnp.bfloat16)
        u.append((x1g, jnp.dot(x1g, lst, preferred_element_type=jnp.float32)))

    for b in range(BB):
        for g in range(NG):
            x1g, u12 = u[b * NG + g]
            u12 = u12.astype(jnp.bfloat16)
            rows = []
            for k in range(K):
                if k == 0:
                    src, off = x1g, 0
                else:
                    src, off = u12, (k - 1) * N
                rows.append(jnp.concatenate(
                    [src[i * Cout:(i + 1) * Cout, off:off + N]
                     for i in range(G)], axis=1))
            ucat = jnp.concatenate(rows, axis=0)
            fg = bfg + jnp.dot(wkcat_ref[...], ucat,
                               preferred_element_type=jnp.float32)
            sig = pl.reciprocal(1.0 + jnp.exp(-fg[Cout:, :]), approx=True)
            res = (fg[:Cout, :] + m[b * NG + g][:Cout, :]) * sig
            for i in range(G):
                o_ref[b, g * G + i] = res[:, i * N:(i + 1) * N].astype(
                    o_ref.dtype)


def kernel(x, supports, conv1_w, conv1_b, conv_1_w, conv_1_b, gcn_w, gcn_b):
    B, Cin, N, T = x.shape
    Cout, _, _, Kt = conv1_w.shape
    K = gcn_w.shape[1] // Cout

    L0 = jnp.eye(N, dtype=supports.dtype)
    L1 = supports
    Ls = [L0, L1]
    for _ in range(2, K):
        L2 = 2.0 * jnp.matmul(L1, L1) - L0
        L0, L1 = L1, L2
        Ls.append(L2)
    Ls = jnp.stack(Ls[1:], axis=0)
    lst = jnp.transpose(Ls, (2, 0, 1)).reshape(N, (K - 1) * N).astype(jnp.bfloat16)

    x_tm = jnp.transpose(x, (0, 3, 1, 2)).reshape(B, T * Cin, N)
    x_tm = x_tm.astype(jnp.bfloat16)

    wt_flat = jnp.transpose(conv1_w[:, :, 0, :], (0, 2, 1)).reshape(Cout, Kt * Cin)
    w1_pad = jnp.concatenate([jnp.zeros((Cout, Cin), conv_1_w.dtype),
                              conv_1_w.reshape(Cout, Cin),
                              jnp.zeros((Cout, Cin), conv_1_w.dtype)], axis=1)
    wcat = jnp.concatenate([w1_pad, wt_flat],
                           axis=0).astype(jnp.bfloat16)
    bcat = jnp.concatenate([conv_1_b, conv1_b]).reshape(2 * Cout, 1)

    wg2 = gcn_w.reshape(2 * Cout, Cout, K)
    wkcat = jnp.transpose(wg2, (0, 2, 1)).reshape(
        2 * Cout, K * Cout).astype(jnp.bfloat16)
    bfg = gcn_b.reshape(2 * Cout, 1)

    cost = pl.CostEstimate(
        flops=2 * B * N * T * (2 * Cout * Kt * Cin + Cout * (K - 1) * N
                               + K * 2 * Cout * Cout),
        transcendentals=B * Cout * N * T,
        bytes_accessed=2 * B * Cin * N * T + 2 * B * Cout * N * T,
    )

    BB = 16
    out_tm = pl.pallas_call(
        lambda *refs: _fused_kernel(T, Cin, Cout, K, N, 4, BB, *refs),
        out_shape=jax.ShapeDtypeStruct((B, T, Cout, N), jnp.bfloat16),
        grid_spec=pl.GridSpec(
            grid=(2, B // BB // 2),
            in_specs=[
                pl.BlockSpec((BB, T * Cin, N),
                             lambda c, s: (c * (B // BB // 2) + s, 0, 0)),
                pl.BlockSpec((2 * Cout, Kt * Cin), lambda c, s: (0, 0)),
                pl.BlockSpec((2 * Cout, 1), lambda c, s: (0, 0)),
                pl.BlockSpec((N, (K - 1) * N), lambda c, s: (0, 0)),
                pl.BlockSpec((2 * Cout, K * Cout), lambda c, s: (0, 0)),
                pl.BlockSpec((2 * Cout, 1), lambda c, s: (0, 0)),
            ],
            out_specs=pl.BlockSpec(
                (BB, T, Cout, N),
                lambda c, s: (c * (B // BB // 2) + s, 0, 0, 0)),
        ),
        compiler_params=pltpu.CompilerParams(
            dimension_semantics=("parallel", "arbitrary")),
        cost_estimate=cost,
    )(x_tm, wcat, bcat, lst, wkcat, bfg)

    return jnp.transpose(out_tm, (0, 2, 3, 1)).astype(jnp.float32)


# speedup vs baseline: 1.0110x; 1.0110x over previous
import jax
import jax.numpy as jnp
from jax.experimental import pallas as pl
from jax.experimental.pallas import tpu as pltpu


def _fused_kernel(T, Cin, Cout, K, N, G, BB,
                  x_ref, wcat_ref, bcat_ref, lst_ref, wkcat_ref, bfg_ref,
                  o_ref):
    wcat = wcat_ref[...]
    bcat = bcat_ref[...]
    lst = lst_ref[...]
    bfg = bfg_ref[...]
    NG = T // G

    m = []
    for b in range(BB):
        xs = x_ref[b]
        zblk = jnp.zeros((Cin, N), xs.dtype)
        for g in range(NG):
            cols = []
            for i in range(G):
                t = g * G + i
                if t == 0:
                    cols.append(jnp.concatenate([zblk, xs[:2 * Cin]], axis=0))
                elif t == T - 1:
                    cols.append(jnp.concatenate([xs[(T - 2) * Cin:], zblk],
                                                axis=0))
                else:
                    cols.append(xs[(t - 1) * Cin:(t + 2) * Cin])
            x3g = jnp.concatenate(cols, axis=1)
            m.append(jnp.dot(wcat, x3g,
                             preferred_element_type=jnp.float32) + bcat)

    u = []
    for mg in m:
        x1g = jnp.concatenate(
            [mg[Cout:, i * N:(i + 1) * N] for i in range(G)],
            axis=0).astype(jnp.bfloat16)
        u.append((x1g, jnp.dot(x1g, lst, preferred_element_type=jnp.float32)))

    for b in range(BB):
        for g in range(NG):
            x1g, u12 = u[b * NG + g]
            u12 = u12.astype(jnp.bfloat16)
            rows = []
            for k in range(K):
                if k == 0:
                    src, off = x1g, 0
                else:
                    src, off = u12, (k - 1) * N
                rows.append(jnp.concatenate(
                    [src[i * Cout:(i + 1) * Cout, off:off + N]
                     for i in range(G)], axis=1))
            ucat = jnp.concatenate(rows, axis=0)
            fg = bfg + jnp.dot(wkcat_ref[...], ucat,
                               preferred_element_type=jnp.float32)
            sig = pl.reciprocal(1.0 + jnp.exp(-fg[Cout:, :]), approx=True)
            res = (fg[:Cout, :] + m[b * NG + g][:Cout, :]) * sig
            for i in range(G):
                o_ref[b, g * G + i] = res[:, i * N:(i + 1) * N].astype(
                    o_ref.dtype)


def kernel(x, supports, conv1_w, conv1_b, conv_1_w, conv_1_b, gcn_w, gcn_b):
    B, Cin, N, T = x.shape
    Cout, _, _, Kt = conv1_w.shape
    K = gcn_w.shape[1] // Cout

    L0 = jnp.eye(N, dtype=supports.dtype)
    L1 = supports
    Ls = [L0, L1]
    for _ in range(2, K):
        L2 = 2.0 * jnp.matmul(L1, L1) - L0
        L0, L1 = L1, L2
        Ls.append(L2)
    Ls = jnp.stack(Ls[1:], axis=0)
    lst = jnp.transpose(Ls, (2, 0, 1)).reshape(N, (K - 1) * N).astype(jnp.bfloat16)

    x_tm = jnp.transpose(x, (0, 3, 1, 2)).reshape(B, T * Cin, N)
    x_tm = x_tm.astype(jnp.bfloat16)

    wt_flat = jnp.transpose(conv1_w[:, :, 0, :], (0, 2, 1)).reshape(Cout, Kt * Cin)
    w1_pad = jnp.concatenate([jnp.zeros((Cout, Cin), conv_1_w.dtype),
                              conv_1_w.reshape(Cout, Cin),
                              jnp.zeros((Cout, Cin), conv_1_w.dtype)], axis=1)
    wcat = jnp.concatenate([w1_pad, wt_flat],
                           axis=0).astype(jnp.bfloat16)
    bcat = jnp.concatenate([conv_1_b, conv1_b]).reshape(2 * Cout, 1)

    wg2 = gcn_w.reshape(2 * Cout, Cout, K)
    wkcat = jnp.transpose(wg2, (0, 2, 1)).reshape(
        2 * Cout, K * Cout).astype(jnp.bfloat16)
    bfg = gcn_b.reshape(2 * Cout, 1)

    cost = pl.CostEstimate(
        flops=2 * B * N * T * (2 * Cout * Kt * Cin + Cout * (K - 1) * N
                               + K * 2 * Cout * Cout),
        transcendentals=B * Cout * N * T,
        bytes_accessed=2 * B * Cin * N * T + 2 * B * Cout * N * T,
    )

    BB = 8
    out_tm = pl.pallas_call(
        lambda *refs: _fused_kernel(T, Cin, Cout, K, N, 4, BB, *refs),
        out_shape=jax.ShapeDtypeStruct((B, T, Cout, N), jnp.bfloat16),
        grid_spec=pl.GridSpec(
            grid=(2, B // BB // 2),
            in_specs=[
                pl.BlockSpec((BB, T * Cin, N),
                             lambda c, s: (c * (B // BB // 2) + s, 0, 0)),
                pl.BlockSpec((2 * Cout, Kt * Cin), lambda c, s: (0, 0)),
                pl.BlockSpec((2 * Cout, 1), lambda c, s: (0, 0)),
                pl.BlockSpec((N, (K - 1) * N), lambda c, s: (0, 0)),
                pl.BlockSpec((2 * Cout, K * Cout), lambda c, s: (0, 0)),
                pl.BlockSpec((2 * Cout, 1), lambda c, s: (0, 0)),
            ],
            out_specs=pl.BlockSpec(
                (BB, T, Cout, N),
                lambda c, s: (c * (B // BB // 2) + s, 0, 0, 0)),
        ),
        compiler_params=pltpu.CompilerParams(
            dimension_semantics=("parallel", "arbitrary")),
        cost_estimate=cost,
    )(x_tm, wcat, bcat, lst, wkcat, bfg)

    return jnp.transpose(out_tm, (0, 2, 3, 1)).astype(jnp.float32)
